# baseline (speedup 1.0000x reference)
import functools

import jax
import jax.numpy as jnp
from jax import lax
from jax.experimental import pallas as pl
from jax.experimental.pallas import tpu as pltpu
from jax.experimental.pallas import tpu_sc as plsc

_NUM_CORES = 2
_NUM_SUBCORES = 16
_NW = _NUM_CORES * _NUM_SUBCORES
_LANES = 16


def _gather_body(chunk, n_chunks, x_hbm, offs_hbm, table_hbm, out_hbm,
                 offs_v, idx_v, rows_v, sem):
    wid = lax.axis_index("s") * _NUM_CORES + lax.axis_index("c")
    base = wid * (chunk * n_chunks)
    pltpu.sync_copy(offs_hbm, offs_v)

    @pl.loop(0, n_chunks)
    def _chunk(g):
        cbase = base + g * chunk
        pltpu.sync_copy(x_hbm.at[pl.ds(cbase, chunk)], idx_v)

        @pl.loop(0, chunk // _LANES, unroll=8)
        def _add(i):
            s = pl.ds(i * _LANES, _LANES)
            idx_v[s] = idx_v[s] + offs_v[s]

        pltpu.async_copy(table_hbm.at[idx_v], rows_v, sem).wait()
        pltpu.sync_copy(rows_v, out_hbm.at[pl.ds(cbase, chunk)])


def kernel(x, table, offsets):
    batch, num_fields = x.shape
    dim = table.shape[1]
    total = batch * num_fields

    chunk = 1664
    assert total % (_NW * chunk) == 0 and chunk % num_fields == 0
    n_chunks = total // (_NW * chunk)

    x_flat = x.reshape(total)
    offs_tile = jnp.tile(offsets.astype(jnp.int32), chunk // num_fields)

    mesh = plsc.VectorSubcoreMesh(core_axis_name="c", subcore_axis_name="s")
    run = pl.kernel(
        functools.partial(_gather_body, chunk, n_chunks),
        out_type=jax.ShapeDtypeStruct((total, dim), table.dtype),
        mesh=mesh,
        scratch_types=[
            pltpu.VMEM((chunk,), jnp.int32),
            pltpu.VMEM((chunk,), jnp.int32),
            pltpu.VMEM((chunk, dim), table.dtype),
            pltpu.SemaphoreType.DMA,
        ],
        compiler_params=pltpu.CompilerParams(use_tc_tiling_on_sc=False),
    )
    out_flat = run(x_flat, offs_tile, table)
    return out_flat.reshape(batch, num_fields, dim)
